# Initial kernel scaffold; baseline (speedup 1.0000x reference)
import jax
import jax.numpy as jnp
from jax.experimental import pallas as pl


def kernel(X, subject_idxs, params):
    raise NotImplementedError("write your pallas kernel here")


# speedup vs baseline: 4.7095x; 4.7095x over previous
import numpy as np
import jax
import jax.numpy as jnp
from jax import lax
from jax.experimental import pallas as pl
from jax.experimental.pallas import tpu as pltpu

HID = 128
EPS = 1e-5
N_SUBJECTS = 4
T = 281
CIN = 271
NC = 1854

NB1 = 8
TCOMP = 43
TPAD = 48
NB2 = 64

_INV_SQRT2 = 0.7071067811865476


def _gelu(x):
    return 0.5 * x * (1.0 + lax.erf(x * _INV_SQRT2))


def _shift_add(d0, d1, d2, bias_row):
    zrow = jnp.zeros((1, d0.shape[1]), jnp.float32)
    return (d1 + bias_row
            + jnp.concatenate([zrow, d0[:-1]], axis=0)
            + jnp.concatenate([d2[1:], zrow], axis=0))


def _conv_tc(z, w3, bias_row):
    d0 = jnp.dot(z, w3[0], preferred_element_type=jnp.float32)
    d1 = jnp.dot(z, w3[1], preferred_element_type=jnp.float32)
    d2 = jnp.dot(z, w3[2], preferred_element_type=jnp.float32)
    return _shift_add(d0, d1, d2, bias_row)


def _p1_kernel(x_ref, w0k_ref, wk_ref, rows_ref, out_ref):
    for b in range(NB1):
        x = x_ref[b]
        dn = (((0,), (0,)), ((), ()))
        d0 = lax.dot_general(x, w0k_ref[0], dn, preferred_element_type=jnp.float32)
        d1 = lax.dot_general(x, w0k_ref[1], dn, preferred_element_type=jnp.float32)
        d2 = lax.dot_general(x, w0k_ref[2], dn, preferred_element_type=jnp.float32)
        y = _shift_add(d0, d1, d2, rows_ref[0])
        a = _gelu(y * rows_ref[1] + rows_ref[2])
        y = _conv_tc(a, wk_ref[0], rows_ref[3]) + a
        h = _gelu(y * rows_ref[4] + rows_ref[5])
        y = _conv_tc(h, wk_ref[1], rows_ref[6]) + h
        a = _gelu(y * rows_ref[7] + rows_ref[8])
        y = _conv_tc(a, wk_ref[2], rows_ref[9]) + a
        h = _gelu(y * rows_ref[10] + rows_ref[11])
        out_ref[b, :] = jnp.sum(h, axis=0)


def _p2_kernel(xsum_ref, subjf_ref, wv_ref, wseg_ref, hk_ref, rows_ref,
               wts_ref, wcls_ref, bcls_ref, out_ref):
    vsum = jnp.dot(xsum_ref[...], wv_ref[...], preferred_element_type=jnp.float32)
    av = jnp.dot(vsum, wseg_ref[...], preferred_element_type=jnp.float32)
    bo = rows_ref[12]

    def seg(s, n):
        col = av[:, s * 128:(s + 1) * 128] + bo
        return jnp.broadcast_to(col[:, None, :], (NB2, n, 128))

    pieces = [seg(0, 9), seg(1, 1), seg(2, 9), seg(3, 1),
              seg(4, 9), seg(5, 1), seg(6, 9)]
    sf = subjf_ref[...]
    for u in range(N_SUBJECTS):
        oh = jnp.where(sf == float(u), 1.0, 0.0)
        pieces.append(oh[:, None, :])
    pieces.append(jnp.zeros((NB2, TPAD - TCOMP, 128), jnp.float32))
    z = jnp.concatenate(pieces, axis=1)
    zf = z.reshape(NB2 * TPAD, 128)

    mask = jnp.tile(jnp.where(wts_ref[...] > 0.0, 1.0, 0.0), (NB2, 1))

    for c in range(2):
        y = _conv_tc(zf, hk_ref[2 * c], rows_ref[6 * c + 0]) + zf
        a = _gelu(y * rows_ref[6 * c + 1] + rows_ref[6 * c + 2]) * mask
        y = _conv_tc(a, hk_ref[2 * c + 1], rows_ref[6 * c + 3]) + a
        zf = _gelu(y * rows_ref[6 * c + 4] + rows_ref[6 * c + 5]) * mask

    zw = zf * jnp.tile(wts_ref[...], (NB2, 1))
    pooled = zw.reshape(NB2, TPAD, 128).sum(axis=1) * (1.0 / 285.0)
    out_ref[...] = (jnp.dot(pooled, wcls_ref[...],
                            preferred_element_type=jnp.float32)
                    + bcls_ref[...])


def _bn_rows(p):
    scale = p['gamma'] / jnp.sqrt(p['var'] + EPS)
    return scale, p['beta'] - p['mean'] * scale


def _block_rows(p):
    s0, o0 = _bn_rows(p['bn0'])
    s1, o1 = _bn_rows(p['bn1'])
    return [p['b0'], s0, o0, p['b1'], s1, o1]


def _seg_fold_consts():
    hfun = [lambda c: 0, lambda c: 0 if c < 32 else 1, lambda c: 1,
            lambda c: 1 if c < 64 else 2, lambda c: 2,
            lambda c: 2 if c < 96 else 3, lambda c: 3]
    S = np.zeros((7, 128, 128), np.float32)
    for s, h in enumerate(hfun):
        for c in range(128):
            S[s, 32 * h(c) + (c % 32), c] += 1.0
    return S


_SEG_S = _seg_fold_consts()

_WTS = []
for _L in (70, 1, 69, 1, 69, 1, 70, 1, 1, 1, 1):
    _WTS += ([1.0] * 4 + [float(_L - 8)] + [1.0] * 4) if _L >= 9 else [1.0] * _L
_WTS += [0.0] * (TPAD - TCOMP)
_WTS = np.asarray(_WTS, np.float32)


def kernel(X, subject_idxs, params):
    B = X.shape[0]
    p = params

    w0k = jnp.transpose(p['block0']['w0'], (2, 1, 0))
    wk = jnp.stack([jnp.transpose(p['block0']['w1'], (2, 1, 0)),
                    jnp.transpose(p['block1']['w0'], (2, 1, 0)),
                    jnp.transpose(p['block1']['w1'], (2, 1, 0))])
    rows1 = jnp.stack(_block_rows(p['block0']) + _block_rows(p['block1']))

    wseg = jnp.einsum('smc,cd->smd', jnp.asarray(_SEG_S), p['wo'])
    wsegf = jnp.concatenate(list(wseg), axis=1)
    hk = jnp.stack([jnp.transpose(p['head0']['w0'], (2, 1, 0)),
                    jnp.transpose(p['head0']['w1'], (2, 1, 0)),
                    jnp.transpose(p['head1']['w0'], (2, 1, 0)),
                    jnp.transpose(p['head1']['w1'], (2, 1, 0))])
    rows2 = jnp.stack(_block_rows(p['head0']) + _block_rows(p['head1'])
                      + [p['bo']])
    wts = jnp.broadcast_to(jnp.asarray(_WTS)[:, None], (TPAD, 128))
    subjf = jnp.broadcast_to(subject_idxs.astype(jnp.float32)[:, None],
                             (B, 128))
    bcls = p['bcls'][None, :]

    xsum = pl.pallas_call(
        _p1_kernel,
        grid=(B // NB1,),
        in_specs=[
            pl.BlockSpec((NB1, CIN, T), lambda i: (i, 0, 0)),
            pl.BlockSpec((3, CIN, 128), lambda i: (0, 0, 0)),
            pl.BlockSpec((3, 3, 128, 128), lambda i: (0, 0, 0, 0)),
            pl.BlockSpec((12, 128), lambda i: (0, 0)),
        ],
        out_specs=pl.BlockSpec((NB1, 128), lambda i: (i, 0)),
        out_shape=jax.ShapeDtypeStruct((B, 128), jnp.float32),
        compiler_params=pltpu.CompilerParams(
            dimension_semantics=("arbitrary",),
            vmem_limit_bytes=100 * 1024 * 1024,
        ),
        name="scband_p1_convs",
    )(X, w0k, wk, rows1)

    out = pl.pallas_call(
        _p2_kernel,
        grid=(B // NB2,),
        in_specs=[
            pl.BlockSpec((NB2, 128), lambda i: (i, 0)),
            pl.BlockSpec((NB2, 128), lambda i: (i, 0)),
            pl.BlockSpec((128, 128), lambda i: (0, 0)),
            pl.BlockSpec((128, 896), lambda i: (0, 0)),
            pl.BlockSpec((4, 3, 128, 128), lambda i: (0, 0, 0, 0)),
            pl.BlockSpec((13, 128), lambda i: (0, 0)),
            pl.BlockSpec((TPAD, 128), lambda i: (0, 0)),
            pl.BlockSpec((128, NC), lambda i: (0, 0)),
            pl.BlockSpec((1, NC), lambda i: (0, 0)),
        ],
        out_specs=pl.BlockSpec((NB2, NC), lambda i: (i, 0)),
        out_shape=jax.ShapeDtypeStruct((B, NC), jnp.float32),
        compiler_params=pltpu.CompilerParams(
            dimension_semantics=("arbitrary",),
            vmem_limit_bytes=100 * 1024 * 1024,
        ),
        name="scband_p2_head",
    )(xsum, subjf, p['wv'], wsegf, hk, rows2, wts, p['wcls'], bcls)

    return out


# speedup vs baseline: 5.3214x; 1.1299x over previous
import numpy as np
import jax
import jax.numpy as jnp
from jax import lax
from jax.experimental import pallas as pl
from jax.experimental.pallas import tpu as pltpu

HID = 128
EPS = 1e-5
N_SUBJECTS = 4
T = 281
CIN = 271
NC = 1854

NB1 = 8
TCOMP = 43
TPAD = 48
NB2 = 64

_INV_SQRT2 = 0.7071067811865476


def _gelu(x):
    return 0.5 * x * (1.0 + lax.erf(x * _INV_SQRT2))


def _shift_add(dcat, bias_row):
    zrow = jnp.zeros((1, 128), jnp.float32)
    return (dcat[:, 128:256] + bias_row
            + jnp.concatenate([zrow, dcat[:-1, 0:128]], axis=0)
            + jnp.concatenate([dcat[1:, 256:384], zrow], axis=0))


def _conv_tc(z, wcat, bias_row):
    dcat = jnp.dot(z, wcat, preferred_element_type=jnp.float32)
    return _shift_add(dcat, bias_row)


def _p1_kernel(x_ref, w0k_ref, wk_ref, rows_ref, out_ref):
    for b in range(NB1):
        x = x_ref[b]
        dn = (((0,), (0,)), ((), ()))
        dcat = lax.dot_general(x, w0k_ref[...], dn,
                               preferred_element_type=jnp.float32)
        y = _shift_add(dcat, rows_ref[0])
        a = _gelu(y * rows_ref[1] + rows_ref[2])
        y = _conv_tc(a, wk_ref[0], rows_ref[3]) + a
        h = _gelu(y * rows_ref[4] + rows_ref[5])
        y = _conv_tc(h, wk_ref[1], rows_ref[6]) + h
        a = _gelu(y * rows_ref[7] + rows_ref[8])
        y = _conv_tc(a, wk_ref[2], rows_ref[9]) + a
        h = _gelu(y * rows_ref[10] + rows_ref[11])
        out_ref[b, :] = jnp.sum(h, axis=0)


def _p2_kernel(xsum_ref, subjf_ref, wv_ref, wseg_ref, hk_ref, rows_ref,
               wts_ref, wcls_ref, bcls_ref, out_ref):
    vsum = jnp.dot(xsum_ref[...], wv_ref[...], preferred_element_type=jnp.float32)
    av = jnp.dot(vsum, wseg_ref[...], preferred_element_type=jnp.float32)
    bo = rows_ref[12]

    def seg(s, n):
        col = av[:, s * 128:(s + 1) * 128] + bo
        return jnp.broadcast_to(col[:, None, :], (NB2, n, 128))

    pieces = [seg(0, 9), seg(1, 1), seg(2, 9), seg(3, 1),
              seg(4, 9), seg(5, 1), seg(6, 9)]
    sf = subjf_ref[...]
    for u in range(N_SUBJECTS):
        oh = jnp.where(sf == float(u), 1.0, 0.0)
        pieces.append(oh[:, None, :])
    pieces.append(jnp.zeros((NB2, TPAD - TCOMP, 128), jnp.float32))
    z = jnp.concatenate(pieces, axis=1)
    zf = z.reshape(NB2 * TPAD, 128)

    mask = jnp.tile(jnp.where(wts_ref[...] > 0.0, 1.0, 0.0), (NB2, 1))

    for c in range(2):
        y = _conv_tc(zf, hk_ref[2 * c], rows_ref[6 * c + 0]) + zf
        a = _gelu(y * rows_ref[6 * c + 1] + rows_ref[6 * c + 2]) * mask
        y = _conv_tc(a, hk_ref[2 * c + 1], rows_ref[6 * c + 3]) + a
        zf = _gelu(y * rows_ref[6 * c + 4] + rows_ref[6 * c + 5]) * mask

    zw = zf * jnp.tile(wts_ref[...], (NB2, 1))
    pooled = zw.reshape(NB2, TPAD, 128).sum(axis=1) * (1.0 / 285.0)
    out_ref[...] = (jnp.dot(pooled, wcls_ref[...],
                            preferred_element_type=jnp.float32)
                    + bcls_ref[...])


def _bn_rows(p):
    scale = p['gamma'] / jnp.sqrt(p['var'] + EPS)
    return scale, p['beta'] - p['mean'] * scale


def _block_rows(p):
    s0, o0 = _bn_rows(p['bn0'])
    s1, o1 = _bn_rows(p['bn1'])
    return [p['b0'], s0, o0, p['b1'], s1, o1]


def _seg_fold_consts():
    hfun = [lambda c: 0, lambda c: 0 if c < 32 else 1, lambda c: 1,
            lambda c: 1 if c < 64 else 2, lambda c: 2,
            lambda c: 2 if c < 96 else 3, lambda c: 3]
    S = np.zeros((7, 128, 128), np.float32)
    for s, h in enumerate(hfun):
        for c in range(128):
            S[s, 32 * h(c) + (c % 32), c] += 1.0
    return S


_SEG_S = _seg_fold_consts()

_WTS = []
for _L in (70, 1, 69, 1, 69, 1, 70, 1, 1, 1, 1):
    _WTS += ([1.0] * 4 + [float(_L - 8)] + [1.0] * 4) if _L >= 9 else [1.0] * _L
_WTS += [0.0] * (TPAD - TCOMP)
_WTS = np.asarray(_WTS, np.float32)


def kernel(X, subject_idxs, params):
    B = X.shape[0]
    p = params

    def _wcat(w):
        return jnp.concatenate([w[:, :, k].T for k in range(3)], axis=1)

    w0k = _wcat(p['block0']['w0'])
    wk = jnp.stack([_wcat(p['block0']['w1']),
                    _wcat(p['block1']['w0']),
                    _wcat(p['block1']['w1'])])
    rows1 = jnp.stack(_block_rows(p['block0']) + _block_rows(p['block1']))

    wseg = jnp.einsum('smc,cd->smd', jnp.asarray(_SEG_S), p['wo'])
    wsegf = jnp.concatenate(list(wseg), axis=1)
    hk = jnp.stack([_wcat(p['head0']['w0']), _wcat(p['head0']['w1']),
                    _wcat(p['head1']['w0']), _wcat(p['head1']['w1'])])
    rows2 = jnp.stack(_block_rows(p['head0']) + _block_rows(p['head1'])
                      + [p['bo']])
    wts = jnp.broadcast_to(jnp.asarray(_WTS)[:, None], (TPAD, 128))
    subjf = jnp.broadcast_to(subject_idxs.astype(jnp.float32)[:, None],
                             (B, 128))
    bcls = p['bcls'][None, :]

    xsum = pl.pallas_call(
        _p1_kernel,
        grid=(B // NB1,),
        in_specs=[
            pl.BlockSpec((NB1, CIN, T), lambda i: (i, 0, 0)),
            pl.BlockSpec((CIN, 384), lambda i: (0, 0)),
            pl.BlockSpec((3, 128, 384), lambda i: (0, 0, 0)),
            pl.BlockSpec((12, 128), lambda i: (0, 0)),
        ],
        out_specs=pl.BlockSpec((NB1, 128), lambda i: (i, 0)),
        out_shape=jax.ShapeDtypeStruct((B, 128), jnp.float32),
        compiler_params=pltpu.CompilerParams(
            dimension_semantics=("arbitrary",),
            vmem_limit_bytes=100 * 1024 * 1024,
        ),
        name="scband_p1_convs",
    )(X, w0k, wk, rows1)

    out = pl.pallas_call(
        _p2_kernel,
        grid=(B // NB2,),
        in_specs=[
            pl.BlockSpec((NB2, 128), lambda i: (i, 0)),
            pl.BlockSpec((NB2, 128), lambda i: (i, 0)),
            pl.BlockSpec((128, 128), lambda i: (0, 0)),
            pl.BlockSpec((128, 896), lambda i: (0, 0)),
            pl.BlockSpec((4, 128, 384), lambda i: (0, 0, 0)),
            pl.BlockSpec((13, 128), lambda i: (0, 0)),
            pl.BlockSpec((TPAD, 128), lambda i: (0, 0)),
            pl.BlockSpec((128, NC), lambda i: (0, 0)),
            pl.BlockSpec((1, NC), lambda i: (0, 0)),
        ],
        out_specs=pl.BlockSpec((NB2, NC), lambda i: (i, 0)),
        out_shape=jax.ShapeDtypeStruct((B, NC), jnp.float32),
        compiler_params=pltpu.CompilerParams(
            dimension_semantics=("arbitrary",),
            vmem_limit_bytes=100 * 1024 * 1024,
        ),
        name="scband_p2_head",
    )(xsum, subjf, p['wv'], wsegf, hk, rows2, wts, p['wcls'], bcls)

    return out
